# baseline (speedup 1.0000x reference)
import functools

import jax
import jax.numpy as jnp
from jax import lax
from jax.experimental import pallas as pl
from jax.experimental.pallas import tpu as pltpu
from jax.experimental.pallas import tpu_sc as plsc

TMG = 256
NW = 32
WS_W = 128


def _router_body(x_ref, wr_ref, meta_ref, cnt_ref, base_ref):
    i = pl.program_id(0)

    @pl.when(i == 0)
    def _():
        base_ref[...] = jnp.zeros_like(base_ref)

    x = x_ref[...]
    wr = wr_ref[...]
    logits = lax.dot_general(
        x, wr, (((1,), (1,)), ((), ())), preferred_element_type=jnp.float32)
    m = jnp.max(logits, axis=-1, keepdims=True)
    ex = jnp.exp(logits - m)
    probs = ex / jnp.sum(ex, axis=-1, keepdims=True)
    tm, e_dim = probs.shape
    ii = lax.broadcasted_iota(jnp.int32, probs.shape, 1)
    m1 = jnp.max(probs, axis=-1, keepdims=True)
    a1 = jnp.min(jnp.where(probs == m1, ii, e_dim), axis=-1, keepdims=True)
    probs2 = jnp.where(ii == a1, -1.0, probs)
    m2 = jnp.max(probs2, axis=-1, keepdims=True)
    a2 = jnp.min(jnp.where(probs2 == m2, ii, e_dim), axis=-1, keepdims=True)

    h = (ii == a1).astype(jnp.float32) + (ii == a2).astype(jnp.float32)
    r = lax.broadcasted_iota(jnp.int32, (tm, tm), 0)
    c = lax.broadcasted_iota(jnp.int32, (tm, tm), 1)
    ltri = (r >= c).astype(jnp.float32)
    incl = lax.dot_general(
        ltri, h, (((1,), (0,)), ((), ())), preferred_element_type=jnp.float32)
    base = base_ref[...]
    rank_mat = base + incl - 1.0
    rank1 = jnp.sum(jnp.where(ii == a1, rank_mat, 0.0), axis=1, keepdims=True)
    rank2 = jnp.sum(jnp.where(ii == a2, rank_mat, 0.0), axis=1, keepdims=True)
    base_new = base + jnp.sum(h, axis=0, keepdims=True)
    base_ref[...] = base_new
    cnt_ref[...] = base_new[None]

    meta = (jnp.where(ii == 0, a1.astype(jnp.float32), 0.0)
            + jnp.where(ii == 1, a2.astype(jnp.float32), 0.0)
            + jnp.where(ii == 2, m1, 0.0)
            + jnp.where(ii == 3, m2, 0.0)
            + jnp.where(ii == 4, rank1, 0.0)
            + jnp.where(ii == 5, rank2, 0.0))
    meta_ref[...] = meta


def _gmlp1_body(te_ref, xs_ref, w1_ref, act_ref):
    del te_ref
    mid = lax.dot_general(
        xs_ref[...], w1_ref[0], (((1,), (1,)), ((), ())),
        preferred_element_type=jnp.float32)
    act_ref[...] = jnp.square(jnp.maximum(mid, 0.0)).astype(jnp.bfloat16)


def _gmlp2_body(te_ref, act_ref, w2_ref, ws_ref, ys_ref):
    del te_ref
    out = lax.dot_general(
        act_ref[...], w2_ref[0], (((1,), (1,)), ((), ())),
        preferred_element_type=jnp.float32)
    ys_ref[...] = out * ws_ref[:, 0:1]


def _make_dispatch(n, c, padn, tpw, ch):
    nch = tpw // ch
    mesh = plsc.VectorSubcoreMesh(core_axis_name="c", subcore_axis_name="s")

    @functools.partial(
        pl.kernel, mesh=mesh,
        out_type=(jax.ShapeDtypeStruct((padn, c), jnp.float32),
                  jax.ShapeDtypeStruct((padn, WS_W), jnp.float32)),
        scratch_types=[
            pltpu.VMEM((ch,), jnp.int32),
            pltpu.VMEM((ch,), jnp.int32),
            pltpu.VMEM((ch, c), jnp.float32),
            pltpu.VMEM((ch, WS_W), jnp.float32),
            pltpu.VMEM((ch, WS_W), jnp.float32),
            pltpu.SemaphoreType.DMA,
            pltpu.SemaphoreType.DMA,
            pltpu.SemaphoreType.DMA,
            pltpu.SemaphoreType.DMA,
        ],
    )
    def dispatch(x_hbm, s0_hbm, s1_hbm, w0_hbm, w1_hbm, xs_hbm, ws_hbm,
                 i0_v, i1_v, rows_v, wa_v, wb_v, sa, sb, sc_, sd):
        wid = lax.axis_index("s") * 2 + lax.axis_index("c")
        base = wid * tpw
        for ci in range(nch):
            st = base + ci * ch
            pltpu.sync_copy(s0_hbm.at[pl.ds(st, ch)], i0_v)
            pltpu.sync_copy(s1_hbm.at[pl.ds(st, ch)], i1_v)
            pltpu.sync_copy(x_hbm.at[pl.ds(st, ch)], rows_v)
            pltpu.sync_copy(w0_hbm.at[pl.ds(st, ch)], wa_v)
            pltpu.sync_copy(w1_hbm.at[pl.ds(st, ch)], wb_v)
            ca = pltpu.async_copy(rows_v, xs_hbm.at[i0_v], sa)
            cb = pltpu.async_copy(rows_v, xs_hbm.at[i1_v], sb)
            cc = pltpu.async_copy(wa_v, ws_hbm.at[i0_v], sc_)
            cd = pltpu.async_copy(wb_v, ws_hbm.at[i1_v], sd)
            ca.wait()
            cb.wait()
            cc.wait()
            cd.wait()

    return dispatch


def _make_combine(n, c, padn, tpw, ch):
    nch = tpw // ch
    nv = c // 16
    mesh = plsc.VectorSubcoreMesh(core_axis_name="c", subcore_axis_name="s")

    @functools.partial(
        pl.kernel, mesh=mesh,
        out_type=jax.ShapeDtypeStruct((n, c), jnp.float32),
        scratch_types=[
            pltpu.VMEM((ch,), jnp.int32),
            pltpu.VMEM((ch,), jnp.int32),
            pltpu.VMEM((ch, c), jnp.float32),
            pltpu.VMEM((ch, c), jnp.float32),
            pltpu.SemaphoreType.DMA,
            pltpu.SemaphoreType.DMA,
        ],
    )
    def combine(ys_hbm, s0_hbm, s1_hbm, out_hbm,
                i0_v, i1_v, y0_v, y1_v, se0, se1):
        wid = lax.axis_index("s") * 2 + lax.axis_index("c")
        base = wid * tpw
        for ci in range(nch):
            st = base + ci * ch
            pltpu.sync_copy(s0_hbm.at[pl.ds(st, ch)], i0_v)
            pltpu.sync_copy(s1_hbm.at[pl.ds(st, ch)], i1_v)
            ga = pltpu.async_copy(ys_hbm.at[i0_v], y0_v, se0)
            gb = pltpu.async_copy(ys_hbm.at[i1_v], y1_v, se1)
            ga.wait()
            gb.wait()

            def tok_body(t, _):
                for vv in range(nv):
                    sl = pl.ds(vv * 16, 16)
                    y0_v[t, sl] = y0_v[t, sl] + y1_v[t, sl]
                return 0

            lax.fori_loop(0, ch, tok_body, 0)
            pltpu.sync_copy(y0_v, out_hbm.at[pl.ds(st, ch)])

    return combine


def kernel(x, Wr, W1, W2):
    b, t, c = x.shape
    n = b * t
    e_dim, dff = W1.shape[0], W1.shape[1]
    x_flat = x.reshape(n, c)
    nt = n * 2 // TMG + e_dim
    padn = nt * TMG

    tm_r = min(512, n)
    n_rt = n // tm_r
    meta, cnt = pl.pallas_call(
        _router_body,
        grid=(n_rt,),
        in_specs=[
            pl.BlockSpec((tm_r, c), lambda i: (i, 0)),
            pl.BlockSpec((e_dim, c), lambda i: (0, 0)),
        ],
        out_specs=[
            pl.BlockSpec((tm_r, 8), lambda i: (i, 0)),
            pl.BlockSpec((1, 1, e_dim), lambda i: (i, 0, 0)),
        ],
        out_shape=[
            jax.ShapeDtypeStruct((n, 8), jnp.float32),
            jax.ShapeDtypeStruct((n_rt, 1, e_dim), jnp.float32),
        ],
        scratch_shapes=[pltpu.VMEM((1, e_dim), jnp.float32)],
        compiler_params=pltpu.CompilerParams(
            dimension_semantics=("arbitrary",),
        ),
    )(x_flat, Wr)

    a1 = meta[:, 0].astype(jnp.int32)
    a2 = meta[:, 1].astype(jnp.int32)
    w1v = meta[:, 2]
    w2v = meta[:, 3]
    rank1 = meta[:, 4].astype(jnp.int32)
    rank2 = meta[:, 5].astype(jnp.int32)
    counts = cnt[-1, 0].astype(jnp.int32)
    padded = ((counts + TMG - 1) // TMG) * TMG
    cum = jnp.cumsum(padded)
    offs = cum - padded
    slot0 = offs[a1] + rank1
    slot1 = offs[a2] + rank2
    tile_starts = jnp.arange(nt, dtype=jnp.int32) * TMG
    tiles_e = jnp.minimum(
        jnp.searchsorted(cum, tile_starts, side="right"),
        e_dim - 1).astype(jnp.int32)
    w0_2d = jnp.broadcast_to(w1v[:, None], (n, WS_W))
    w1_2d = jnp.broadcast_to(w2v[:, None], (n, WS_W))

    tpw = n // NW
    xs, ws = _make_dispatch(n, c, padn, tpw, 32)(
        x_flat, slot0, slot1, w0_2d, w1_2d)

    grid_spec1 = pltpu.PrefetchScalarGridSpec(
        num_scalar_prefetch=1,
        grid=(nt,),
        in_specs=[
            pl.BlockSpec((TMG, c), lambda i, te: (i, 0)),
            pl.BlockSpec((1, dff, c), lambda i, te: (te[i], 0, 0)),
        ],
        out_specs=pl.BlockSpec((TMG, dff), lambda i, te: (i, 0)),
    )
    act = pl.pallas_call(
        _gmlp1_body,
        grid_spec=grid_spec1,
        out_shape=jax.ShapeDtypeStruct((padn, dff), jnp.bfloat16),
        compiler_params=pltpu.CompilerParams(
            dimension_semantics=("arbitrary",),
        ),
    )(tiles_e, xs, W1)
    grid_spec2 = pltpu.PrefetchScalarGridSpec(
        num_scalar_prefetch=1,
        grid=(nt,),
        in_specs=[
            pl.BlockSpec((TMG, dff), lambda i, te: (i, 0)),
            pl.BlockSpec((1, c, dff), lambda i, te: (te[i], 0, 0)),
            pl.BlockSpec((TMG, WS_W), lambda i, te: (i, 0)),
        ],
        out_specs=pl.BlockSpec((TMG, c), lambda i, te: (i, 0)),
    )
    ys = pl.pallas_call(
        _gmlp2_body,
        grid_spec=grid_spec2,
        out_shape=jax.ShapeDtypeStruct((padn, c), jnp.float32),
        compiler_params=pltpu.CompilerParams(
            dimension_semantics=("arbitrary",),
        ),
    )(tiles_e, act, W2, ws)

    out = _make_combine(n, c, padn, tpw, 16)(ys, slot0, slot1)
    return out.reshape(b, t, c)


# speedup vs baseline: 1.0102x; 1.0102x over previous
import functools

import jax
import jax.numpy as jnp
from jax import lax
from jax.experimental import pallas as pl
from jax.experimental.pallas import tpu as pltpu
from jax.experimental.pallas import tpu_sc as plsc

TMG = 256
NW = 32
WS_W = 128


def _router_body(x_ref, wr_ref, meta_ref, cnt_ref, base_ref):
    i = pl.program_id(0)

    @pl.when(i == 0)
    def _():
        base_ref[...] = jnp.zeros_like(base_ref)

    x = x_ref[...]
    wr = wr_ref[...]
    logits = lax.dot_general(
        x, wr, (((1,), (1,)), ((), ())), preferred_element_type=jnp.float32)
    m = jnp.max(logits, axis=-1, keepdims=True)
    ex = jnp.exp(logits - m)
    probs = ex / jnp.sum(ex, axis=-1, keepdims=True)
    tm, e_dim = probs.shape
    ii = lax.broadcasted_iota(jnp.int32, probs.shape, 1)
    m1 = jnp.max(probs, axis=-1, keepdims=True)
    a1 = jnp.min(jnp.where(probs == m1, ii, e_dim), axis=-1, keepdims=True)
    probs2 = jnp.where(ii == a1, -1.0, probs)
    m2 = jnp.max(probs2, axis=-1, keepdims=True)
    a2 = jnp.min(jnp.where(probs2 == m2, ii, e_dim), axis=-1, keepdims=True)

    h = (ii == a1).astype(jnp.float32) + (ii == a2).astype(jnp.float32)
    r = lax.broadcasted_iota(jnp.int32, (tm, tm), 0)
    c = lax.broadcasted_iota(jnp.int32, (tm, tm), 1)
    ltri = (r >= c).astype(jnp.float32)
    incl = lax.dot_general(
        ltri, h, (((1,), (0,)), ((), ())), preferred_element_type=jnp.float32)
    base = base_ref[...]
    rank_mat = base + incl - 1.0
    rank1 = jnp.sum(jnp.where(ii == a1, rank_mat, 0.0), axis=1, keepdims=True)
    rank2 = jnp.sum(jnp.where(ii == a2, rank_mat, 0.0), axis=1, keepdims=True)
    base_new = base + jnp.sum(h, axis=0, keepdims=True)
    base_ref[...] = base_new
    cnt_ref[...] = base_new[None]

    meta = (jnp.where(ii == 0, a1.astype(jnp.float32), 0.0)
            + jnp.where(ii == 1, a2.astype(jnp.float32), 0.0)
            + jnp.where(ii == 2, m1, 0.0)
            + jnp.where(ii == 3, m2, 0.0)
            + jnp.where(ii == 4, rank1, 0.0)
            + jnp.where(ii == 5, rank2, 0.0))
    meta_ref[...] = meta


def _gmlp_body(te_ref, xs_ref, w1_ref, w2_ref, ws_ref, ys_ref):
    del te_ref
    mid = lax.dot_general(
        xs_ref[...], w1_ref[0], (((1,), (1,)), ((), ())),
        preferred_element_type=jnp.float32)
    act = jnp.square(jnp.maximum(mid, 0.0))
    out = lax.dot_general(
        act, w2_ref[0], (((1,), (1,)), ((), ())),
        preferred_element_type=jnp.float32)
    ys_ref[...] = out * ws_ref[:, 0:1]


def _make_dispatch(n, c, padn, tpw, ch):
    nch = tpw // ch
    mesh = plsc.VectorSubcoreMesh(core_axis_name="c", subcore_axis_name="s")

    @functools.partial(
        pl.kernel, mesh=mesh,
        out_type=(jax.ShapeDtypeStruct((padn, c), jnp.float32),
                  jax.ShapeDtypeStruct((padn, WS_W), jnp.float32)),
        scratch_types=[
            pltpu.VMEM((ch,), jnp.int32),
            pltpu.VMEM((ch,), jnp.int32),
            pltpu.VMEM((ch, c), jnp.float32),
            pltpu.VMEM((ch, WS_W), jnp.float32),
            pltpu.VMEM((ch, WS_W), jnp.float32),
            pltpu.SemaphoreType.DMA,
            pltpu.SemaphoreType.DMA,
            pltpu.SemaphoreType.DMA,
            pltpu.SemaphoreType.DMA,
        ],
    )
    def dispatch(x_hbm, s0_hbm, s1_hbm, w0_hbm, w1_hbm, xs_hbm, ws_hbm,
                 i0_v, i1_v, rows_v, wa_v, wb_v, sa, sb, sc_, sd):
        wid = lax.axis_index("s") * 2 + lax.axis_index("c")
        base = wid * tpw
        for ci in range(nch):
            st = base + ci * ch
            pltpu.sync_copy(s0_hbm.at[pl.ds(st, ch)], i0_v)
            pltpu.sync_copy(s1_hbm.at[pl.ds(st, ch)], i1_v)
            pltpu.sync_copy(x_hbm.at[pl.ds(st, ch)], rows_v)
            pltpu.sync_copy(w0_hbm.at[pl.ds(st, ch)], wa_v)
            pltpu.sync_copy(w1_hbm.at[pl.ds(st, ch)], wb_v)
            ca = pltpu.async_copy(rows_v, xs_hbm.at[i0_v], sa)
            cb = pltpu.async_copy(rows_v, xs_hbm.at[i1_v], sb)
            cc = pltpu.async_copy(wa_v, ws_hbm.at[i0_v], sc_)
            cd = pltpu.async_copy(wb_v, ws_hbm.at[i1_v], sd)
            ca.wait()
            cb.wait()
            cc.wait()
            cd.wait()

    return dispatch


def _make_combine(n, c, padn, tpw, ch):
    nch = tpw // ch
    nv = c // 16
    mesh = plsc.VectorSubcoreMesh(core_axis_name="c", subcore_axis_name="s")

    @functools.partial(
        pl.kernel, mesh=mesh,
        out_type=jax.ShapeDtypeStruct((n, c), jnp.float32),
        scratch_types=[
            pltpu.VMEM((ch,), jnp.int32),
            pltpu.VMEM((ch,), jnp.int32),
            pltpu.VMEM((ch, c), jnp.float32),
            pltpu.VMEM((ch, c), jnp.float32),
            pltpu.SemaphoreType.DMA,
            pltpu.SemaphoreType.DMA,
        ],
    )
    def combine(ys_hbm, s0_hbm, s1_hbm, out_hbm,
                i0_v, i1_v, y0_v, y1_v, se0, se1):
        wid = lax.axis_index("s") * 2 + lax.axis_index("c")
        base = wid * tpw
        for ci in range(nch):
            st = base + ci * ch
            pltpu.sync_copy(s0_hbm.at[pl.ds(st, ch)], i0_v)
            pltpu.sync_copy(s1_hbm.at[pl.ds(st, ch)], i1_v)
            ga = pltpu.async_copy(ys_hbm.at[i0_v], y0_v, se0)
            gb = pltpu.async_copy(ys_hbm.at[i1_v], y1_v, se1)
            ga.wait()
            gb.wait()

            def tok_body(t, _):
                for vv in range(nv):
                    sl = pl.ds(vv * 16, 16)
                    y0_v[t, sl] = y0_v[t, sl] + y1_v[t, sl]
                return 0

            lax.fori_loop(0, ch, tok_body, 0)
            pltpu.sync_copy(y0_v, out_hbm.at[pl.ds(st, ch)])

    return combine


def kernel(x, Wr, W1, W2):
    b, t, c = x.shape
    n = b * t
    e_dim, dff = W1.shape[0], W1.shape[1]
    x_flat = x.reshape(n, c)
    nt = n * 2 // TMG + e_dim
    padn = nt * TMG

    tm_r = min(512, n)
    n_rt = n // tm_r
    meta, cnt = pl.pallas_call(
        _router_body,
        grid=(n_rt,),
        in_specs=[
            pl.BlockSpec((tm_r, c), lambda i: (i, 0)),
            pl.BlockSpec((e_dim, c), lambda i: (0, 0)),
        ],
        out_specs=[
            pl.BlockSpec((tm_r, 8), lambda i: (i, 0)),
            pl.BlockSpec((1, 1, e_dim), lambda i: (i, 0, 0)),
        ],
        out_shape=[
            jax.ShapeDtypeStruct((n, 8), jnp.float32),
            jax.ShapeDtypeStruct((n_rt, 1, e_dim), jnp.float32),
        ],
        scratch_shapes=[pltpu.VMEM((1, e_dim), jnp.float32)],
        compiler_params=pltpu.CompilerParams(
            dimension_semantics=("arbitrary",),
        ),
    )(x_flat, Wr)

    a1 = meta[:, 0].astype(jnp.int32)
    a2 = meta[:, 1].astype(jnp.int32)
    w1v = meta[:, 2]
    w2v = meta[:, 3]
    rank1 = meta[:, 4].astype(jnp.int32)
    rank2 = meta[:, 5].astype(jnp.int32)
    counts = cnt[-1, 0].astype(jnp.int32)
    padded = ((counts + TMG - 1) // TMG) * TMG
    cum = jnp.cumsum(padded)
    offs = cum - padded
    slot0 = offs[a1] + rank1
    slot1 = offs[a2] + rank2
    tile_starts = jnp.arange(nt, dtype=jnp.int32) * TMG
    tiles_e = jnp.minimum(
        jnp.searchsorted(cum, tile_starts, side="right"),
        e_dim - 1).astype(jnp.int32)
    w0_2d = jnp.broadcast_to(w1v[:, None], (n, WS_W))
    w1_2d = jnp.broadcast_to(w2v[:, None], (n, WS_W))

    tpw = n // NW
    xs, ws = _make_dispatch(n, c, padn, tpw, 32)(
        x_flat, slot0, slot1, w0_2d, w1_2d)

    grid_spec = pltpu.PrefetchScalarGridSpec(
        num_scalar_prefetch=1,
        grid=(nt,),
        in_specs=[
            pl.BlockSpec((TMG, c), lambda i, te: (i, 0)),
            pl.BlockSpec((1, dff, c), lambda i, te: (te[i], 0, 0),
                         pipeline_mode=pl.Buffered(buffer_count=1)),
            pl.BlockSpec((1, c, dff), lambda i, te: (te[i], 0, 0),
                         pipeline_mode=pl.Buffered(buffer_count=1)),
            pl.BlockSpec((TMG, WS_W), lambda i, te: (i, 0)),
        ],
        out_specs=pl.BlockSpec((TMG, c), lambda i, te: (i, 0)),
    )
    ys = pl.pallas_call(
        _gmlp_body,
        grid_spec=grid_spec,
        out_shape=jax.ShapeDtypeStruct((padn, c), jnp.float32),
        compiler_params=pltpu.CompilerParams(
            dimension_semantics=("arbitrary",),
        ),
    )(tiles_e, xs, W1, W2, ws)

    out = _make_combine(n, c, padn, tpw, 16)(ys, slot0, slot1)
    return out.reshape(b, t, c)
